# baseline (speedup 1.0000x reference)
import jax
import jax.numpy as jnp
from jax.experimental import pallas as pl

_R = 1000


def _heads_body(x_ref, cons_ref, obj_ref):
    i = pl.program_id(0)
    xb = x_ref[:]
    m = jnp.max(xb, axis=1, keepdims=True)
    e = jnp.exp(xb - m)
    s = jnp.sum(e, axis=1, keepdims=True)
    cons_ref[:] = e / s
    t = jax.nn.sigmoid(xb.T)
    for k in range(pl.num_programs(0)):
        @pl.when(i == k)
        def _():
            obj_ref[:, k * _R:(k + 1) * _R] = t


def kernel(x, graph, edge_index):
    del graph, edge_index
    n, d = x.shape
    r = _R if n % _R == 0 else n
    cons, obj = pl.pallas_call(
        _heads_body,
        grid=(n // r,),
        in_specs=[pl.BlockSpec((r, d), lambda i: (i, 0))],
        out_specs=[
            pl.BlockSpec((r, d), lambda i: (i, 0)),
            pl.BlockSpec((d, n), lambda i: (0, 0)),
        ],
        out_shape=[
            jax.ShapeDtypeStruct((n, d), x.dtype),
            jax.ShapeDtypeStruct((d, n), x.dtype),
        ],
    )(x)
    return (cons, obj)


# speedup vs baseline: 1.2669x; 1.2669x over previous
import jax
import jax.numpy as jnp
from jax.experimental import pallas as pl


def _heads_body(x_ref, cons_ref, obj_ref):
    xb = x_ref[:]
    m = jnp.max(xb, axis=1, keepdims=True)
    e = jnp.exp(xb - m)
    s = jnp.sum(e, axis=1, keepdims=True)
    cons_ref[:] = e / s
    obj_ref[:] = jax.nn.sigmoid(xb.T)


def kernel(x, graph, edge_index):
    del graph, edge_index
    n, d = x.shape
    cons, obj = pl.pallas_call(
        _heads_body,
        out_shape=[
            jax.ShapeDtypeStruct((n, d), x.dtype),
            jax.ShapeDtypeStruct((d, n), x.dtype),
        ],
    )(x)
    return (cons, obj)
